# Initial kernel scaffold; baseline (speedup 1.0000x reference)
import jax
import jax.numpy as jnp
from jax.experimental import pallas as pl


def kernel(inputs, addr_emb, pc_emb, Wa1, ba1, Wa2, ba2, Wp1, bp1, Wp2, bp2):
    raise NotImplementedError("write your pallas kernel here")


# speedup vs baseline: 3.2797x; 3.2797x over previous
import functools

import jax
import jax.numpy as jnp
from jax import lax
from jax.experimental import pallas as pl
from jax.experimental.pallas import tpu as pltpu
from jax.experimental.pallas import tpu_sc as plsc

B = 16384
NTOK = 8 * B
NW = 32
TPW = NTOK // NW
L = 16


def _table_body(ae, pe, wa1, ba1, wa2, ba2, wp1, bp1, wp2, bp2, out_ref):
    ha = jnp.maximum(
        jnp.dot(ae[...], wa1[...], preferred_element_type=jnp.float32)
        + ba1[...], 0.0)
    oa = jnp.maximum(
        jnp.dot(ha, wa2[...], preferred_element_type=jnp.float32)
        + ba2[...], 0.0)
    hp = jnp.maximum(
        jnp.dot(pe[...], wp1[...], preferred_element_type=jnp.float32)
        + bp1[...], 0.0)
    op = jnp.maximum(
        jnp.dot(hp, wp2[...], preferred_element_type=jnp.float32)
        + bp2[...], 0.0)
    out_ref[0:256, :] = oa
    out_ref[256:512, :] = op


_table_call = pl.pallas_call(
    _table_body,
    out_shape=jax.ShapeDtypeStruct((512, 2), jnp.float32),
)


@functools.partial(
    pl.kernel,
    out_type=jax.ShapeDtypeStruct((2 * NTOK,), jnp.float32),
    mesh=plsc.VectorSubcoreMesh(core_axis_name="c", subcore_axis_name="s"),
    scratch_types=[
        pltpu.VMEM((TPW,), jnp.int32),
        pltpu.VMEM((1024,), jnp.float32),
        pltpu.VMEM((2 * TPW,), jnp.float32),
    ],
    compiler_params=pltpu.CompilerParams(needs_layout_passes=False),
)
def _gather_call(idx_hbm, table_hbm, out_hbm, idx_v, table_v, out_v):
    wid = lax.axis_index("s") * 2 + lax.axis_index("c")
    out_off = wid * TPW
    in_off = lax.rem(out_off + 4 * B, NTOK)
    fbase = jnp.where(wid < NW // 2, 0, 512).astype(jnp.int32)

    pltpu.sync_copy(table_hbm, table_v)
    pltpu.sync_copy(idx_hbm.at[pl.ds(in_off, TPW)], idx_v)

    lane = lax.iota(jnp.int32, L)

    def body(i, carry):
        fi = idx_v[pl.ds(i * L, L)] * 2 + fbase
        pos = (lane + i * L) * 2
        v0 = plsc.load_gather(table_v, [fi])
        v1 = plsc.load_gather(table_v, [fi + 1])
        plsc.store_scatter(out_v, [pos], v0)
        plsc.store_scatter(out_v, [pos + 1], v1)
        return carry

    lax.fori_loop(0, TPW // L, body, jnp.int32(0))

    pltpu.sync_copy(out_v, out_hbm.at[pl.ds(2 * out_off, 2 * TPW)])


def kernel(inputs, addr_emb, pc_emb, Wa1, ba1, Wa2, ba2, Wp1, bp1, Wp2, bp2):
    table = _table_call(
        addr_emb, pc_emb,
        Wa1, ba1.reshape(1, 8), Wa2, ba2.reshape(1, 2),
        Wp1, bp1.reshape(1, 8), Wp2, bp2.reshape(1, 2))
    idx_flat = inputs.reshape(-1)
    out_flat = _gather_call(idx_flat, table.reshape(-1))
    return out_flat.reshape(NTOK, 2)


# speedup vs baseline: 13.4728x; 4.1079x over previous
import functools

import jax
import jax.numpy as jnp
from jax import lax
from jax.experimental import pallas as pl
from jax.experimental.pallas import tpu as pltpu
from jax.experimental.pallas import tpu_sc as plsc

B = 16384
NTOK = 8 * B
NW = 32
TPW = NTOK // NW
L = 16


def _table_body(ae, pe, wa1, ba1, wa2, ba2, wp1, bp1, wp2, bp2, out_ref):
    ha = jnp.maximum(
        jnp.dot(ae[...], wa1[...], preferred_element_type=jnp.float32)
        + ba1[...], 0.0)
    oa = jnp.maximum(
        jnp.dot(ha, wa2[...], preferred_element_type=jnp.float32)
        + ba2[...], 0.0)
    hp = jnp.maximum(
        jnp.dot(pe[...], wp1[...], preferred_element_type=jnp.float32)
        + bp1[...], 0.0)
    op = jnp.maximum(
        jnp.dot(hp, wp2[...], preferred_element_type=jnp.float32)
        + bp2[...], 0.0)
    out_ref[0:256, :] = oa
    out_ref[256:512, :] = op


_table_call = pl.pallas_call(
    _table_body,
    out_shape=jax.ShapeDtypeStruct((512, 2), jnp.float32),
)


@functools.partial(
    pl.kernel,
    out_type=jax.ShapeDtypeStruct((2 * NTOK,), jnp.float32),
    mesh=plsc.VectorSubcoreMesh(core_axis_name="c", subcore_axis_name="s"),
    scratch_types=[
        pltpu.VMEM((TPW,), jnp.int32),
        pltpu.VMEM((1024,), jnp.float32),
        pltpu.VMEM((2 * TPW,), jnp.float32),
    ],
    compiler_params=pltpu.CompilerParams(needs_layout_passes=False),
)
def _gather_call(idx_hbm, table_hbm, out_hbm, idx_v, table_v, out_v):
    wid = lax.axis_index("s") * 2 + lax.axis_index("c")
    out_off = wid * TPW
    in_off = lax.rem(out_off + 4 * B, NTOK)
    fbase = jnp.where(wid < NW // 2, 0, 512).astype(jnp.int32)

    pltpu.sync_copy(table_hbm, table_v)
    pltpu.sync_copy(idx_hbm.at[pl.ds(in_off, TPW)], idx_v)

    def block(b, carry):
        for s in range(8):
            fi = idx_v[pl.ds(b * 128 + s * L, L)] * 2 + fbase
            v0 = plsc.load_gather(table_v, [fi])
            v1 = plsc.load_gather(table_v, [fi + 1])
            out_v[pl.ds(b * 256 + s * L, L)] = v0
            out_v[pl.ds(b * 256 + 128 + s * L, L)] = v1
        return carry

    lax.fori_loop(0, TPW // 128, block, jnp.int32(0))

    pltpu.sync_copy(out_v, out_hbm.at[pl.ds(2 * out_off, 2 * TPW)])


def kernel(inputs, addr_emb, pc_emb, Wa1, ba1, Wa2, ba2, Wp1, bp1, Wp2, bp2):
    table = _table_call(
        addr_emb, pc_emb,
        Wa1, ba1.reshape(1, 8), Wa2, ba2.reshape(1, 2),
        Wp1, bp1.reshape(1, 8), Wp2, bp2.reshape(1, 2))
    idx_flat = inputs.reshape(-1)
    out_flat = _gather_call(idx_flat, table.reshape(-1))
    return out_flat.reshape(NTOK // 128, 2, 128).transpose(0, 2, 1).reshape(NTOK, 2)


# speedup vs baseline: 17.4785x; 1.2973x over previous
import functools

import jax
import jax.numpy as jnp
from jax import lax
from jax.experimental import pallas as pl
from jax.experimental.pallas import tpu as pltpu
from jax.experimental.pallas import tpu_sc as plsc

B = 16384
NTOK = 8 * B
NW = 32
TPW = NTOK // NW
L = 16


def _table_body(aeT, peT, wa1T, ba1, wa2T, ba2, wp1T, bp1, wp2T, bp2, out_ref):
    ba1c = jnp.transpose(ba1[...], (1, 0))
    ba2c = jnp.transpose(ba2[...], (1, 0))
    bp1c = jnp.transpose(bp1[...], (1, 0))
    bp2c = jnp.transpose(bp2[...], (1, 0))
    haT = jnp.maximum(
        jnp.dot(wa1T[...], aeT[...], preferred_element_type=jnp.float32)
        + ba1c, 0.0)
    oaT = jnp.maximum(
        jnp.dot(wa2T[...], haT, preferred_element_type=jnp.float32)
        + ba2c, 0.0)
    hpT = jnp.maximum(
        jnp.dot(wp1T[...], peT[...], preferred_element_type=jnp.float32)
        + bp1c, 0.0)
    opT = jnp.maximum(
        jnp.dot(wp2T[...], hpT, preferred_element_type=jnp.float32)
        + bp2c, 0.0)
    allT = jnp.concatenate([oaT, opT], axis=1)
    out_ref[...] = allT.reshape(8, 128)


_table_call = pl.pallas_call(
    _table_body,
    out_shape=jax.ShapeDtypeStruct((8, 128), jnp.float32),
)


@functools.partial(
    pl.kernel,
    out_type=jax.ShapeDtypeStruct((2 * NTOK,), jnp.float32),
    mesh=plsc.VectorSubcoreMesh(core_axis_name="c", subcore_axis_name="s"),
    scratch_types=[
        pltpu.VMEM((TPW,), jnp.int32),
        pltpu.VMEM((1024,), jnp.float32),
        pltpu.VMEM((2 * TPW,), jnp.float32),
    ],
    compiler_params=pltpu.CompilerParams(needs_layout_passes=False),
)
def _gather_call(idx_hbm, table_hbm, out_hbm, idx_v, table_v, out_v):
    wid = lax.axis_index("s") * 2 + lax.axis_index("c")
    out_off = wid * TPW
    in_off = lax.rem(out_off + 4 * B, NTOK)
    pbase = jnp.where(wid < NW // 2, 0, 256).astype(jnp.int32)

    pltpu.sync_copy(table_hbm, table_v)
    pltpu.sync_copy(idx_hbm.at[pl.ds(in_off, TPW)], idx_v)

    def block(b, carry):
        for s in range(8):
            fi = idx_v[pl.ds(b * 128 + s * L, L)] + pbase
            v0 = plsc.load_gather(table_v, [fi])
            v1 = plsc.load_gather(table_v, [fi + 512])
            out_v[pl.ds(b * 256 + s * L, L)] = v0
            out_v[pl.ds(b * 256 + 128 + s * L, L)] = v1
        return carry

    lax.fori_loop(0, TPW // 128, block, jnp.int32(0))

    pltpu.sync_copy(out_v, out_hbm.at[pl.ds(2 * out_off, 2 * TPW)])


def kernel(inputs, addr_emb, pc_emb, Wa1, ba1, Wa2, ba2, Wp1, bp1, Wp2, bp2):
    table = _table_call(
        addr_emb.T, pc_emb.T,
        Wa1.T, ba1.reshape(1, 8), Wa2.T, ba2.reshape(1, 2),
        Wp1.T, bp1.reshape(1, 8), Wp2.T, bp2.reshape(1, 2))
    idx_flat = inputs.reshape(-1)
    out_flat = _gather_call(idx_flat, table.reshape(-1))
    return out_flat.reshape(NTOK // 128, 2, 128).transpose(0, 2, 1).reshape(NTOK, 2)
